# initial kernel scaffold (unmeasured)
import jax
import jax.numpy as jnp
from jax import lax
from jax.experimental import pallas as pl
from jax.experimental.pallas import tpu as pltpu

N_DEV = 4


def kernel(x, w_mat, scale_x, scale_w):
    m_tot, k_loc = x.shape
    k_tot, n = w_mat.shape
    m_loc = m_tot // N_DEV

    def body(x_ref, w_ref, sx_ref, sw_ref, out_ref, xfull_ref,
             send_sems, recv_sems):
        my = lax.axis_index("i")

        barrier_sem = pltpu.get_barrier_semaphore()
        for off in range(1, N_DEV):
            pl.semaphore_signal(
                barrier_sem, inc=1,
                device_id=((my + off) % N_DEV,),
                device_id_type=pl.DeviceIdType.MESH,
            )
        pl.semaphore_wait(barrier_sem, N_DEV - 1)

        xfull_ref[:, pl.ds(my * k_loc, k_loc)] = x_ref[pl.ds(my * m_loc, m_loc), :]

        rdmas = []
        for off in range(1, N_DEV):
            dst = (my + off) % N_DEV
            rdma = pltpu.make_async_remote_copy(
                src_ref=x_ref.at[pl.ds(dst * m_loc, m_loc), :],
                dst_ref=xfull_ref.at[:, pl.ds(my * k_loc, k_loc)],
                send_sem=send_sems.at[off - 1],
                recv_sem=recv_sems.at[off - 1],
                device_id=(dst,),
                device_id_type=pl.DeviceIdType.MESH,
            )
            rdma.start()
            rdmas.append(rdma)
        for rdma in rdmas:
            rdma.wait()

        acc = jnp.dot(xfull_ref[:, :], w_ref[:, :],
                      preferred_element_type=jnp.float32)
        y = acc * (sx_ref[0] * sw_ref[0])
        yc = jnp.clip(y, -60.0, 60.0)
        out_ref[:, :] = y / (1.0 + jnp.exp(-yc))

    return pl.pallas_call(
        body,
        out_shape=jax.ShapeDtypeStruct((m_loc, n), jnp.float32),
        in_specs=[
            pl.BlockSpec(memory_space=pltpu.VMEM),
            pl.BlockSpec(memory_space=pltpu.VMEM),
            pl.BlockSpec(memory_space=pltpu.SMEM),
            pl.BlockSpec(memory_space=pltpu.SMEM),
        ],
        out_specs=pl.BlockSpec(memory_space=pltpu.VMEM),
        scratch_shapes=[
            pltpu.VMEM((m_loc, k_tot), x.dtype),
            pltpu.SemaphoreType.DMA((N_DEV - 1,)),
            pltpu.SemaphoreType.DMA((N_DEV - 1,)),
        ],
        compiler_params=pltpu.CompilerParams(collective_id=0),
    )(x, w_mat, scale_x, scale_w)


# baseline (device time: 117858 ns/iter reference)
import jax
import jax.numpy as jnp
from jax import lax
from jax.experimental import pallas as pl
from jax.experimental.pallas import tpu as pltpu

N_DEV = 4
N_BLK = 512


def kernel(x, w_mat, scale_x, scale_w):
    m_tot, k_loc = x.shape
    k_tot, n = w_mat.shape
    m_loc = m_tot // N_DEV
    n_steps = n // N_BLK

    def body(x_ref, w_ref, sx_ref, sw_ref, out_ref,
             xsend_ref, xfull_ref, send_sems, recv_sems):
        step = pl.program_id(0)
        my = lax.axis_index("i")

        @pl.when(step == 0)
        def _():
            barrier_sem = pltpu.get_barrier_semaphore()
            for off in range(1, N_DEV):
                pl.semaphore_signal(
                    barrier_sem, inc=1,
                    device_id=((my + off) % N_DEV,),
                    device_id_type=pl.DeviceIdType.MESH,
                )
            pl.semaphore_wait(barrier_sem, N_DEV - 1)

            xsend_ref[:, :] = x_ref[:, :].astype(jnp.float8_e4m3fn)
            xfull_ref[:, pl.ds(my * k_loc, k_loc)] = (
                xsend_ref[pl.ds(my * m_loc, m_loc), :])

            rdmas = []
            for off in range(1, N_DEV):
                dst = (my + off) % N_DEV
                rdma = pltpu.make_async_remote_copy(
                    src_ref=xsend_ref.at[pl.ds(dst * m_loc, m_loc), :],
                    dst_ref=xfull_ref.at[:, pl.ds(my * k_loc, k_loc)],
                    send_sem=send_sems.at[off - 1],
                    recv_sem=recv_sems.at[off - 1],
                    device_id=(dst,),
                    device_id_type=pl.DeviceIdType.MESH,
                )
                rdma.start()
                rdmas.append(rdma)
            for rdma in rdmas:
                rdma.wait()

        wf8 = w_ref[:, :].astype(jnp.float8_e4m3fn)
        acc = jnp.dot(xfull_ref[:, :], wf8,
                      preferred_element_type=jnp.float32)
        y = acc * (sx_ref[0] * sw_ref[0])
        out_ref[:, :] = y / (1.0 + jnp.exp(-jnp.clip(y, -60.0, 60.0)))

    return pl.pallas_call(
        body,
        grid=(n_steps,),
        out_shape=jax.ShapeDtypeStruct((m_loc, n), jnp.float32),
        in_specs=[
            pl.BlockSpec((m_tot, k_loc), lambda i: (0, 0)),
            pl.BlockSpec((k_tot, N_BLK), lambda i: (0, i)),
            pl.BlockSpec(memory_space=pltpu.SMEM),
            pl.BlockSpec(memory_space=pltpu.SMEM),
        ],
        out_specs=pl.BlockSpec((m_loc, N_BLK), lambda i: (0, i)),
        scratch_shapes=[
            pltpu.VMEM((m_tot, k_loc), jnp.float8_e4m3fn),
            pltpu.VMEM((m_loc, k_tot), jnp.float8_e4m3fn),
            pltpu.SemaphoreType.DMA((N_DEV - 1,)),
            pltpu.SemaphoreType.DMA((N_DEV - 1,)),
        ],
        compiler_params=pltpu.CompilerParams(
            collective_id=0,
            dimension_semantics=("arbitrary",),
        ),
    )(x, w_mat, scale_x, scale_w)


# device time: 105791 ns/iter; 1.1141x vs baseline; 1.1141x over previous
import jax
import jax.numpy as jnp
from jax import lax
from jax.experimental import pallas as pl
from jax.experimental.pallas import tpu as pltpu

N_DEV = 4
N_BLK = 512
S = 3


def kernel(x, w_mat, scale_x, scale_w):
    m_tot, k_loc = x.shape
    k_tot, n = w_mat.shape
    m_loc = m_tot // N_DEV
    n_steps = n // N_BLK

    def body(x_ref, w_hbm, sx_ref, sw_ref, out_ref,
             xsend_ref, xfull_ref, wbuf_ref, send_sems, recv_sems, wsems):
        step = pl.program_id(0)
        my = lax.axis_index("i")

        @pl.when(step == 0)
        def _():
            for s in range(S):
                pltpu.make_async_copy(
                    w_hbm.at[:, pl.ds(s * N_BLK, N_BLK)],
                    wbuf_ref.at[s], wsems.at[s],
                ).start()

            barrier_sem = pltpu.get_barrier_semaphore()
            for off in range(1, N_DEV):
                pl.semaphore_signal(
                    barrier_sem, inc=1,
                    device_id=((my + off) % N_DEV,),
                    device_id_type=pl.DeviceIdType.MESH,
                )
            pl.semaphore_wait(barrier_sem, N_DEV - 1)

            xsend_ref[:, :] = x_ref[:, :].astype(jnp.float8_e4m3fn)
            xfull_ref[:, pl.ds(my * k_loc, k_loc)] = (
                xsend_ref[pl.ds(my * m_loc, m_loc), :])

            rdmas = []
            for off in range(1, N_DEV):
                dst = (my + off) % N_DEV
                rdma = pltpu.make_async_remote_copy(
                    src_ref=xsend_ref.at[pl.ds(dst * m_loc, m_loc), :],
                    dst_ref=xfull_ref.at[:, pl.ds(my * k_loc, k_loc)],
                    send_sem=send_sems.at[off - 1],
                    recv_sem=recv_sems.at[off - 1],
                    device_id=(dst,),
                    device_id_type=pl.DeviceIdType.MESH,
                )
                rdma.start()
                rdmas.append(rdma)
            for rdma in rdmas:
                rdma.wait()

        slot = lax.rem(step, S)
        pltpu.make_async_copy(
            w_hbm.at[:, pl.ds(step * N_BLK, N_BLK)],
            wbuf_ref.at[slot], wsems.at[slot],
        ).wait()

        wf8 = wbuf_ref[slot].astype(jnp.float8_e4m3fn)
        acc = jnp.dot(xfull_ref[:, :], wf8,
                      preferred_element_type=jnp.float32)
        y = acc * (sx_ref[0] * sw_ref[0])
        out_ref[:, :] = y / (1.0 + jnp.exp(-jnp.clip(y, -60.0, 60.0)))

        @pl.when(step + S < n_steps)
        def _():
            pltpu.make_async_copy(
                w_hbm.at[:, pl.ds((step + S) * N_BLK, N_BLK)],
                wbuf_ref.at[slot], wsems.at[slot],
            ).start()

    return pl.pallas_call(
        body,
        grid=(n_steps,),
        out_shape=jax.ShapeDtypeStruct((m_loc, n), jnp.float32),
        in_specs=[
            pl.BlockSpec((m_tot, k_loc), lambda i: (0, 0)),
            pl.BlockSpec(memory_space=pl.ANY),
            pl.BlockSpec(memory_space=pltpu.SMEM),
            pl.BlockSpec(memory_space=pltpu.SMEM),
        ],
        out_specs=pl.BlockSpec((m_loc, N_BLK), lambda i: (0, i)),
        scratch_shapes=[
            pltpu.VMEM((m_tot, k_loc), jnp.float8_e4m3fn),
            pltpu.VMEM((m_loc, k_tot), jnp.float8_e4m3fn),
            pltpu.VMEM((S, k_tot, N_BLK), jnp.float32),
            pltpu.SemaphoreType.DMA((N_DEV - 1,)),
            pltpu.SemaphoreType.DMA((N_DEV - 1,)),
            pltpu.SemaphoreType.DMA((S,)),
        ],
        compiler_params=pltpu.CompilerParams(
            collective_id=0,
            dimension_semantics=("arbitrary",),
            vmem_limit_bytes=62 * 1024 * 1024,
        ),
    )(x, w_mat, scale_x, scale_w)
